# baseline (speedup 1.0000x reference)
import functools

import jax
import jax.numpy as jnp
from jax import lax
from jax.experimental import pallas as pl
from jax.experimental.pallas import tpu as pltpu
from jax.experimental.pallas import tpu_sc as plsc

EPS = 1e-05
_N = 65536
_D = 64
_NW = 32
_RAYS_PER_W = _N // _NW
_CHUNK = 1024
_NCHUNK = _RAYS_PER_W // _CHUNK


def _sc_body(w_hbm, o_hbm, buf, sem_i, sem_o):
    cid = lax.axis_index("c")
    sid = lax.axis_index("s")
    wid = sid * 2 + cid
    base = wid * _RAYS_PER_W

    def compute():
        def col_group(j, carry):
            sl = pl.ds(j * 16, 16)
            s = buf[0, sl]
            for c in range(1, _D):
                s = s + buf[c, sl]
            pad = jnp.maximum(EPS - s, 0.0)
            inv = 1.0 / (s + pad)
            a = pad * (1.0 / _D)
            for c in range(_D):
                buf[c, sl] = (buf[c, sl] + a) * inv
            return carry

        lax.fori_loop(0, _CHUNK // 16, col_group, 0)

    for g in range(_NCHUNK):
        sl = pl.ds(base + g * _CHUNK, _CHUNK)
        pltpu.async_copy(w_hbm.at[:, sl], buf, sem_i).wait()
        compute()
        pltpu.async_copy(buf, o_hbm.at[:, sl], sem_o).wait()


@functools.cache
def _sc_pdf():
    mesh = plsc.VectorSubcoreMesh(core_axis_name="c", subcore_axis_name="s")
    return pl.kernel(
        _sc_body,
        out_type=jax.ShapeDtypeStruct((_D, _N), jnp.float32),
        mesh=mesh,
        scratch_types=[
            pltpu.VMEM((_D, _CHUNK), jnp.float32),
            pltpu.SemaphoreType.DMA,
            pltpu.SemaphoreType.DMA,
        ],
    )


def kernel(weights, stratified):
    wt = weights.T
    out_t = _sc_pdf()(wt)
    return out_t.T


# speedup vs baseline: 4.3309x; 4.3309x over previous
import jax
import jax.numpy as jnp
from jax.experimental import pallas as pl
from jax.experimental.pallas import tpu as pltpu

EPS = 1e-05
_BLOCK_COLS = 32768


def _pdf_block(w_ref, o_ref):
    w = w_ref[...]
    s = jnp.sum(w, axis=0, keepdims=True)
    pad = jnp.maximum(EPS - s, 0.0)
    inv = 1.0 / (s + pad)
    o_ref[...] = (w + pad * (1.0 / w.shape[0])) * inv


def kernel(weights, stratified):
    n, d = weights.shape
    wt = weights.T
    out_t = pl.pallas_call(
        _pdf_block,
        grid=(n // _BLOCK_COLS,),
        in_specs=[pl.BlockSpec((d, _BLOCK_COLS), lambda i: (0, i))],
        out_specs=pl.BlockSpec((d, _BLOCK_COLS), lambda i: (0, i)),
        out_shape=jax.ShapeDtypeStruct((d, n), weights.dtype),
        compiler_params=pltpu.CompilerParams(
            dimension_semantics=("parallel",),
        ),
    )(wt)
    return out_t.T
